# baseline (speedup 1.0000x reference)
import functools
import jax
import jax.numpy as jnp
from jax import lax
from jax.experimental import pallas as pl
from jax.experimental.pallas import tpu as pltpu
from jax.experimental.pallas import tpu_sc as plsc

N = 10000
E = 320000
D = 128

NC = 2
NS = 16
NW = NC * NS
K = 32
NCHUNKS = E // K
NSLOT = (NCHUNKS + NW - 1) // NW
NVEC = K // 16
ROWS_PER_FLUSH = N // 10
DEN_PER_FLUSH = N // 5
ZROWS = 40


def _tables_body(h_ref, wh_ref, we_ref, wp_ref, wa_ref,
                 zh_ref, ps_ref, pd_ref, as_ref, ad_ref, wep_ref, wea_ref):
    f32 = jnp.float32
    zh = jnp.dot(h_ref[...], wh_ref[...], preferred_element_type=f32)
    zh_ref[...] = zh
    wp = wp_ref[...]
    wa = wa_ref[...]
    ps_ref[...] = jnp.dot(zh, wp[D:2 * D], preferred_element_type=f32)
    pd_ref[...] = jnp.dot(zh, wp[2 * D:], preferred_element_type=f32)
    as_ref[...] = jnp.dot(zh, wa[D:2 * D], preferred_element_type=f32)
    ad_ref[...] = jnp.dot(zh, wa[2 * D:], preferred_element_type=f32)
    we = we_ref[...]
    wep_ref[...] = jnp.dot(we, wp[:D], preferred_element_type=f32)
    wea_ref[...] = jnp.dot(we, wa[:D], preferred_element_type=f32)


def _edge_dense_body(e_ref, wep_ref, wea_ref, pe_ref, ae_ref):
    f32 = jnp.float32
    eb = e_ref[...]
    pe_ref[...] = jnp.dot(eb, wep_ref[...], preferred_element_type=f32)
    ae_ref[...] = jnp.dot(eb, wea_ref[...], preferred_element_type=f32)


def _sc_body(src_hbm, dst_hbm, ae_hbm, pe_hbm, zh_hbm, ps_hbm, pd_hbm,
             as_hbm, ad_hbm,
             eproj_hbm, den_hbm, u_hbm, csum_hbm, cssq_hbm,
             srcv0, dstv0, aev0, asg0, adg0, exv0, pev0, zrows0, psrows0,
             pdrows0,
             srcv1, dstv1, aev1, asg1, adg1, exv1, pev1, zrows1, psrows1,
             pdrows1,
             csumv, cssqv, dzv, zbuf,
             u_sh, den_sh, isem0, isem1, gsem0, gsem1, osem0, osem1,
             asem0, asem1):
    c = lax.axis_index("c")
    s = lax.axis_index("s")
    wid = s * NC + c
    zvec = jnp.zeros((16,), jnp.float32)
    bufs = (
        dict(srcv=srcv0, dstv=dstv0, aev=aev0, asg=asg0, adg=adg0, exv=exv0,
             pev=pev0, zrows=zrows0, psrows=psrows0, pdrows=pdrows0,
             isem=isem0, gsem=gsem0, osem=osem0, asem=asem0),
        dict(srcv=srcv1, dstv=dstv1, aev=aev1, asg=asg1, adg=adg1, exv=exv1,
             pev=pev1, zrows=zrows1, psrows=psrows1, pdrows=pdrows1,
             isem=isem1, gsem=gsem1, osem=osem1, asem=asem1),
    )

    for kk in range(D // 16):
        csumv[pl.ds(kk * 16, 16)] = zvec
        cssqv[pl.ds(kk * 16, 16)] = zvec

    def zrow_body(r, _):
        for kk in range(D // 16):
            zbuf[r, pl.ds(kk * 16, 16)] = zvec
        return 0

    lax.fori_loop(0, ZROWS, zrow_body, 0)

    @pl.when(s < 10)
    def _zero_u():
        rb = s * ROWS_PER_FLUSH

        def zcp_body(j, _):
            pltpu.sync_copy(zbuf, u_sh.at[pl.ds(rb + j * ZROWS, ZROWS)])
            return 0

        lax.fori_loop(0, ROWS_PER_FLUSH // ZROWS, zcp_body, 0)

    for kk in range(DEN_PER_FLUSH // 16):
        dzv[pl.ds(kk * 16, 16)] = zvec

    @pl.when(s < 5)
    def _zero_den():
        rb = s * DEN_PER_FLUSH
        pltpu.sync_copy(dzv, den_sh.at[pl.ds(rb, DEN_PER_FLUSH)])

    plsc.subcore_barrier()

    def cbase(t):
        return (wid + NW * t) * K

    def valid(t):
        return jnp.logical_and(t <= NSLOT - 1, wid + NW * t < NCHUNKS)

    def issue_idx(t, B):
        b = cbase(t)
        pltpu.async_copy(src_hbm.at[pl.ds(b, K)], B["srcv"], B["isem"])
        pltpu.async_copy(dst_hbm.at[pl.ds(b, K)], B["dstv"], B["isem"])
        pltpu.async_copy(ae_hbm.at[pl.ds(b, K)], B["aev"], B["isem"])

    def wait_idx(B):
        pltpu.make_async_copy(src_hbm.at[pl.ds(0, K)], B["srcv"], B["isem"]).wait()
        pltpu.make_async_copy(dst_hbm.at[pl.ds(0, K)], B["dstv"], B["isem"]).wait()
        pltpu.make_async_copy(ae_hbm.at[pl.ds(0, K)], B["aev"], B["isem"]).wait()

    def issue_gathers(t, B):
        b = cbase(t)
        pltpu.async_copy(pe_hbm.at[pl.ds(b, K)], B["pev"], B["gsem"])
        pltpu.async_copy(zh_hbm.at[B["srcv"]], B["zrows"], B["gsem"])
        pltpu.async_copy(ps_hbm.at[B["srcv"]], B["psrows"], B["gsem"])
        pltpu.async_copy(pd_hbm.at[B["dstv"]], B["pdrows"], B["gsem"])
        pltpu.async_copy(as_hbm.at[B["srcv"]], B["asg"], B["gsem"])
        pltpu.async_copy(ad_hbm.at[B["dstv"]], B["adg"], B["gsem"])

    def wait_gathers(B):
        pltpu.make_async_copy(pe_hbm.at[pl.ds(0, K)], B["pev"], B["gsem"]).wait()
        pltpu.make_async_copy(zh_hbm.at[B["srcv"]], B["zrows"], B["gsem"]).wait()
        pltpu.make_async_copy(ps_hbm.at[B["srcv"]], B["psrows"], B["gsem"]).wait()
        pltpu.make_async_copy(pd_hbm.at[B["dstv"]], B["pdrows"], B["gsem"]).wait()
        pltpu.make_async_copy(as_hbm.at[B["srcv"]], B["asg"], B["gsem"]).wait()
        pltpu.make_async_copy(ad_hbm.at[B["dstv"]], B["adg"], B["gsem"]).wait()

    def issue_outs(t, B):
        b = cbase(t)
        pltpu.async_copy(B["pev"], eproj_hbm.at[pl.ds(b, K)], B["osem"])
        h_den = pltpu.async_copy(B["exv"], den_sh.at[B["dstv"]], B["asem"],
                                 add=True)
        h_u = pltpu.async_copy(B["zrows"], u_sh.at[B["dstv"]], B["asem"],
                               add=True)
        return h_den, h_u

    def wait_outs(t, B):
        b = cbase(t)
        pltpu.make_async_copy(B["pev"], eproj_hbm.at[pl.ds(b, K)], B["osem"]).wait()

    def compute(B):
        for g in range(NVEC):
            sl = pl.ds(g * 16, 16)
            att = jnp.maximum(B["aev"][sl] + B["asg"][sl] + B["adg"][sl], 0.0)
            B["exv"][sl] = jnp.exp(att)
        for g in range(NVEC):
            exvec = B["exv"][pl.ds(g * 16, 16)]
            for j in range(16):
                i = g * 16 + j
                ex_i = exvec[j]
                for kk in range(D // 16):
                    sl = pl.ds(kk * 16, 16)
                    B["zrows"][i, sl] = B["zrows"][i, sl] * ex_i
                    v = (B["pev"][i, sl] + B["psrows"][i, sl]
                         + B["pdrows"][i, sl])
                    B["pev"][i, sl] = v
                    csumv[sl] = csumv[sl] + v
                    cssqv[sl] = cssqv[sl] + v * v

    pltpu.sync_copy(src_hbm.at[pl.ds(cbase(0), K)], bufs[0]["srcv"])
    pltpu.sync_copy(dst_hbm.at[pl.ds(cbase(0), K)], bufs[0]["dstv"])
    pltpu.sync_copy(ae_hbm.at[pl.ds(cbase(0), K)], bufs[0]["aev"])
    issue_gathers(0, bufs[0])

    def pair_body(jp, carry):
        for bb in range(2):
            t = jp * 2 + bb
            B = bufs[bb]
            Bn = bufs[1 - bb]

            handles = []

            @pl.when(jnp.logical_and(t >= 1, valid(t - 1)))
            def _wo():
                wait_outs(t - 1, Bn)

            @pl.when(valid(t))
            def _cp():
                wait_gathers(B)
                compute(B)
                handles.append(issue_outs(t, B))

            @pl.when(valid(t + 1))
            def _pf():
                issue_idx(t + 1, Bn)
                wait_idx(Bn)
                issue_gathers(t + 1, Bn)

            @pl.when(valid(t))
            def _wa():
                h_den, h_u = handles[0]
                h_den.wait()
                h_u.wait()
        return 0

    lax.fori_loop(0, (NSLOT + 1) // 2, pair_body, 0)
    plsc.subcore_barrier()

    @pl.when(s < 10)
    def _flush_u():
        rb = s * ROWS_PER_FLUSH
        pltpu.sync_copy(u_sh.at[pl.ds(rb, ROWS_PER_FLUSH)],
                        u_hbm.at[c, pl.ds(rb, ROWS_PER_FLUSH)])

    @pl.when(s < 5)
    def _flush_den():
        rb = s * DEN_PER_FLUSH
        pltpu.sync_copy(den_sh.at[pl.ds(rb, DEN_PER_FLUSH)], dzv)
        pltpu.sync_copy(dzv, den_hbm.at[pl.ds(c * N + rb, DEN_PER_FLUSH)])

    pltpu.sync_copy(csumv, csum_hbm.at[wid])
    pltpu.sync_copy(cssqv, cssq_hbm.at[wid])


def _finalize_h_body(u_ref, den_ref, csum_ref, cssq_ref,
                     gh_ref, bh_ref, ge_ref, be_ref,
                     hout_ref, estats_ref):
    u = u_ref[0] + u_ref[1]
    den = den_ref[0] + den_ref[1]
    den = jnp.where(den == 0.0, 1.0, den)
    h_new = u / den[:, None]
    mu = jnp.mean(h_new, axis=0)
    var = jnp.mean((h_new - mu) ** 2, axis=0)
    inv = gh_ref[...] / jnp.sqrt(var + 1e-5)
    hout_ref[...] = jnp.maximum((h_new - mu) * inv + bh_ref[...], 0.0)
    cs = jnp.sum(csum_ref[...], axis=0)
    css = jnp.sum(cssq_ref[...], axis=0)
    mu_e = cs / E
    var_e = css / E - mu_e * mu_e
    scale = ge_ref[...] / jnp.sqrt(var_e + 1e-5)
    shift = be_ref[...] - mu_e * scale
    estats_ref[0, :] = scale
    estats_ref[1, :] = shift


def _finalize_e_body(ep_ref, st_ref, eout_ref):
    eout_ref[...] = jnp.maximum(ep_ref[...] * st_ref[0, :] + st_ref[1, :], 0.0)


def kernel(h, e, edge_index, W_h, W_e, W_p, b_p, W_a,
           gamma_h, beta_h, gamma_e, beta_e):
    f32 = jnp.float32
    sds = jax.ShapeDtypeStruct

    z_h, P_s, P_d, a_s2, a_d2, W_ep, w_ea = pl.pallas_call(
        _tables_body,
        out_shape=[sds((N, D), f32), sds((N, D), f32), sds((N, D), f32),
                   sds((N, 1), f32), sds((N, 1), f32),
                   sds((D, D), f32), sds((D, 1), f32)],
    )(h, W_h, W_e, W_p, W_a)

    BE = 4000
    P_e, a_e2 = pl.pallas_call(
        _edge_dense_body,
        grid=(E // BE,),
        in_specs=[pl.BlockSpec((BE, D), lambda i: (i, 0)),
                  pl.BlockSpec((D, D), lambda i: (0, 0)),
                  pl.BlockSpec((D, 1), lambda i: (0, 0))],
        out_specs=[pl.BlockSpec((BE, D), lambda i: (i, 0)),
                   pl.BlockSpec((BE, 1), lambda i: (i, 0))],
        out_shape=[sds((E, D), f32), sds((E, 1), f32)],
    )(e, W_ep, w_ea)

    src = edge_index[0]
    dst = edge_index[1]
    a_e = a_e2.reshape(E)
    a_s = a_s2.reshape(N)
    a_d = a_d2.reshape(N)

    mesh = plsc.VectorSubcoreMesh(core_axis_name="c", subcore_axis_name="s")
    sc_fn = pl.kernel(
        _sc_body,
        out_type=[sds((E, D), f32), sds((NC * N,), f32), sds((NC, N, D), f32),
                  sds((NW, D), f32), sds((NW, D), f32)],
        mesh=mesh,
        scratch_types=(
            [pltpu.VMEM((K,), jnp.int32),
             pltpu.VMEM((K,), jnp.int32),
             pltpu.VMEM((K,), f32),
             pltpu.VMEM((K,), f32),
             pltpu.VMEM((K,), f32),
             pltpu.VMEM((K,), f32),
             pltpu.VMEM((K, D), f32),
             pltpu.VMEM((K, D), f32),
             pltpu.VMEM((K, D), f32),
             pltpu.VMEM((K, D), f32),
             ] * 2
            + [
                pltpu.VMEM((D,), f32),
                pltpu.VMEM((D,), f32),
                pltpu.VMEM((DEN_PER_FLUSH,), f32),
                pltpu.VMEM((ZROWS, D), f32),
                pltpu.VMEM_SHARED((N, D), f32),
                pltpu.VMEM_SHARED((N,), f32),
            ]
            + [pltpu.SemaphoreType.DMA] * 8
        ),
    )
    e_proj, den2, u2, csum, cssq = sc_fn(
        src, dst, a_e, P_e, z_h, P_s, P_d, a_s, a_d)
    den2 = den2.reshape(NC, N)

    h_out, estats = pl.pallas_call(
        _finalize_h_body,
        out_shape=[sds((N, D), f32), sds((2, D), f32)],
    )(u2, den2, csum, cssq, gamma_h, beta_h, gamma_e, beta_e)

    e_out = pl.pallas_call(
        _finalize_e_body,
        grid=(E // BE,),
        in_specs=[pl.BlockSpec((BE, D), lambda i: (i, 0)),
                  pl.BlockSpec((2, D), lambda i: (0, 0))],
        out_specs=pl.BlockSpec((BE, D), lambda i: (i, 0)),
        out_shape=sds((E, D), f32),
    )(e_proj, estats)

    return h_out, e_out


# speedup vs baseline: 1.2434x; 1.2434x over previous
import functools
import jax
import jax.numpy as jnp
from jax import lax
from jax.experimental import pallas as pl
from jax.experimental.pallas import tpu as pltpu
from jax.experimental.pallas import tpu_sc as plsc

N = 10000
E = 320000
D = 128

NC = 2
NS = 16
NW = NC * NS
K = 32
NCHUNKS = E // K
NSLOT = (NCHUNKS + NW - 1) // NW
NVEC = K // 16
ROWS_PER_FLUSH = N // 10
DEN_PER_FLUSH = N // 5
ZROWS = 40


def _tables_body(h_ref, wh_ref, we_ref, wp_ref, wa_ref,
                 zh_ref, ps_ref, pd_ref, as_ref, ad_ref, wep_ref, wea_ref):
    f32 = jnp.float32
    zh = jnp.dot(h_ref[...], wh_ref[...], preferred_element_type=f32)
    zh_ref[...] = zh
    wp = wp_ref[...]
    wa = wa_ref[...]
    ps_ref[...] = jnp.dot(zh, wp[D:2 * D], preferred_element_type=f32)
    pd_ref[...] = jnp.dot(zh, wp[2 * D:], preferred_element_type=f32)
    as_ref[...] = jnp.dot(zh, wa[D:2 * D], preferred_element_type=f32)
    ad_ref[...] = jnp.dot(zh, wa[2 * D:], preferred_element_type=f32)
    we = we_ref[...]
    wep_ref[...] = jnp.dot(we, wp[:D], preferred_element_type=f32)
    wea_ref[...] = jnp.dot(we, wa[:D], preferred_element_type=f32)


def _edge_dense_body(e_ref, wep_ref, wea_ref, pe_ref, ae_ref):
    f32 = jnp.float32
    eb = e_ref[...]
    pe_ref[...] = jnp.dot(eb, wep_ref[...], preferred_element_type=f32)
    ae_ref[...] = jnp.dot(eb, wea_ref[...], preferred_element_type=f32)


def _sc_body(src_hbm, dst_hbm, ae_hbm, pe_hbm, zh_hbm, ps_hbm, pd_hbm,
             as_hbm, ad_hbm,
             eproj_hbm, den_hbm, u_hbm, csum_hbm, cssq_hbm,
             srcv0, dstv0, aev0, asg0, adg0, exv0, pev0, zrows0, psrows0,
             pdrows0,
             srcv1, dstv1, aev1, asg1, adg1, exv1, pev1, zrows1, psrows1,
             pdrows1,
             csumv, cssqv, dzv, zbuf,
             u_sh, den_sh, isem0, isem1, gsem0, gsem1, osem0, osem1,
             asem0, asem1):
    c = lax.axis_index("c")
    s = lax.axis_index("s")
    wid = s * NC + c
    zvec = jnp.zeros((16,), jnp.float32)
    bufs = (
        dict(srcv=srcv0, dstv=dstv0, aev=aev0, asg=asg0, adg=adg0, exv=exv0,
             pev=pev0, zrows=zrows0, psrows=psrows0, pdrows=pdrows0,
             isem=isem0, gsem=gsem0, osem=osem0, asem=asem0),
        dict(srcv=srcv1, dstv=dstv1, aev=aev1, asg=asg1, adg=adg1, exv=exv1,
             pev=pev1, zrows=zrows1, psrows=psrows1, pdrows=pdrows1,
             isem=isem1, gsem=gsem1, osem=osem1, asem=asem1),
    )

    for kk in range(D // 16):
        csumv[pl.ds(kk * 16, 16)] = zvec
        cssqv[pl.ds(kk * 16, 16)] = zvec

    def zrow_body(r, _):
        for kk in range(D // 16):
            zbuf[r, pl.ds(kk * 16, 16)] = zvec
        return 0

    lax.fori_loop(0, ZROWS, zrow_body, 0)

    @pl.when(s < 10)
    def _zero_u():
        rb = s * ROWS_PER_FLUSH

        def zcp_body(j, _):
            pltpu.sync_copy(zbuf, u_sh.at[pl.ds(rb + j * ZROWS, ZROWS)])
            return 0

        lax.fori_loop(0, ROWS_PER_FLUSH // ZROWS, zcp_body, 0)

    for kk in range(DEN_PER_FLUSH // 16):
        dzv[pl.ds(kk * 16, 16)] = zvec

    @pl.when(s < 5)
    def _zero_den():
        rb = s * DEN_PER_FLUSH
        pltpu.sync_copy(dzv, den_sh.at[pl.ds(rb, DEN_PER_FLUSH)])

    plsc.subcore_barrier()

    def cbase(t):
        return (wid + NW * t) * K

    def valid(t):
        return jnp.logical_and(t <= NSLOT - 1, wid + NW * t < NCHUNKS)

    def issue_idx(t, B):
        b = cbase(t)
        pltpu.async_copy(src_hbm.at[pl.ds(b, K)], B["srcv"], B["isem"])
        pltpu.async_copy(dst_hbm.at[pl.ds(b, K)], B["dstv"], B["isem"])
        pltpu.async_copy(ae_hbm.at[pl.ds(b, K)], B["aev"], B["isem"])

    def wait_idx(B):
        pltpu.make_async_copy(src_hbm.at[pl.ds(0, K)], B["srcv"], B["isem"]).wait()
        pltpu.make_async_copy(dst_hbm.at[pl.ds(0, K)], B["dstv"], B["isem"]).wait()
        pltpu.make_async_copy(ae_hbm.at[pl.ds(0, K)], B["aev"], B["isem"]).wait()

    def issue_gathers(t, B):
        b = cbase(t)
        pltpu.async_copy(pe_hbm.at[pl.ds(b, K)], B["pev"], B["gsem"])
        pltpu.async_copy(zh_hbm.at[B["srcv"]], B["zrows"], B["gsem"])
        pltpu.async_copy(ps_hbm.at[B["srcv"]], B["psrows"], B["gsem"])
        pltpu.async_copy(pd_hbm.at[B["dstv"]], B["pdrows"], B["gsem"])
        pltpu.async_copy(as_hbm.at[B["srcv"]], B["asg"], B["gsem"])
        pltpu.async_copy(ad_hbm.at[B["dstv"]], B["adg"], B["gsem"])

    def wait_gathers(B):
        pltpu.make_async_copy(pe_hbm.at[pl.ds(0, K)], B["pev"], B["gsem"]).wait()
        pltpu.make_async_copy(zh_hbm.at[B["srcv"]], B["zrows"], B["gsem"]).wait()
        pltpu.make_async_copy(ps_hbm.at[B["srcv"]], B["psrows"], B["gsem"]).wait()
        pltpu.make_async_copy(pd_hbm.at[B["dstv"]], B["pdrows"], B["gsem"]).wait()
        pltpu.make_async_copy(as_hbm.at[B["srcv"]], B["asg"], B["gsem"]).wait()
        pltpu.make_async_copy(ad_hbm.at[B["dstv"]], B["adg"], B["gsem"]).wait()

    def issue_outs(t, B):
        b = cbase(t)
        pltpu.async_copy(B["pev"], eproj_hbm.at[pl.ds(b, K)], B["osem"])
        h_den = pltpu.async_copy(B["exv"], den_sh.at[B["dstv"]], B["asem"],
                                 add=True)
        h_u = pltpu.async_copy(B["zrows"], u_sh.at[B["dstv"]], B["asem"],
                               add=True)
        return h_den, h_u

    def wait_outs(t, B):
        b = cbase(t)
        pltpu.make_async_copy(B["pev"], eproj_hbm.at[pl.ds(b, K)], B["osem"]).wait()

    def compute(B):
        for g in range(NVEC):
            sl = pl.ds(g * 16, 16)
            att = jnp.maximum(B["aev"][sl] + B["asg"][sl] + B["adg"][sl], 0.0)
            B["exv"][sl] = jnp.exp(att)
        for g in range(NVEC):
            exvec = B["exv"][pl.ds(g * 16, 16)]
            for j in range(16):
                i = g * 16 + j
                ex_i = exvec[j]
                for kk in range(D // 16):
                    sl = pl.ds(kk * 16, 16)
                    B["zrows"][i, sl] = B["zrows"][i, sl] * ex_i
                    v = (B["pev"][i, sl] + B["psrows"][i, sl]
                         + B["pdrows"][i, sl])
                    B["pev"][i, sl] = v
                    csumv[sl] = csumv[sl] + v
                    cssqv[sl] = cssqv[sl] + v * v

    pltpu.sync_copy(src_hbm.at[pl.ds(cbase(0), K)], bufs[0]["srcv"])
    pltpu.sync_copy(dst_hbm.at[pl.ds(cbase(0), K)], bufs[0]["dstv"])
    pltpu.sync_copy(ae_hbm.at[pl.ds(cbase(0), K)], bufs[0]["aev"])
    issue_gathers(0, bufs[0])

    def pair_body(jp, carry):
        for bb in range(2):
            t = jp * 2 + bb
            B = bufs[bb]
            Bn = bufs[1 - bb]

            handles = []

            @pl.when(jnp.logical_and(t >= 1, valid(t - 1)))
            def _wo():
                wait_outs(t - 1, Bn)

            @pl.when(valid(t + 1))
            def _pf_idx():
                issue_idx(t + 1, Bn)

            @pl.when(valid(t))
            def _wg():
                wait_gathers(B)

            @pl.when(valid(t + 1))
            def _pf_g():
                wait_idx(Bn)
                issue_gathers(t + 1, Bn)

            @pl.when(valid(t))
            def _cp():
                compute(B)
                handles.append(issue_outs(t, B))

            @pl.when(valid(t))
            def _wa():
                h_den, h_u = handles[0]
                h_den.wait()
                h_u.wait()
        return 0

    lax.fori_loop(0, (NSLOT + 1) // 2, pair_body, 0)
    plsc.subcore_barrier()

    @pl.when(s < 10)
    def _flush_u():
        rb = s * ROWS_PER_FLUSH
        pltpu.sync_copy(u_sh.at[pl.ds(rb, ROWS_PER_FLUSH)],
                        u_hbm.at[c, pl.ds(rb, ROWS_PER_FLUSH)])

    @pl.when(s < 5)
    def _flush_den():
        rb = s * DEN_PER_FLUSH
        pltpu.sync_copy(den_sh.at[pl.ds(rb, DEN_PER_FLUSH)], dzv)
        pltpu.sync_copy(dzv, den_hbm.at[pl.ds(c * N + rb, DEN_PER_FLUSH)])

    pltpu.sync_copy(csumv, csum_hbm.at[wid])
    pltpu.sync_copy(cssqv, cssq_hbm.at[wid])


def _finalize_h_body(u_ref, den_ref, csum_ref, cssq_ref,
                     gh_ref, bh_ref, ge_ref, be_ref,
                     hout_ref, estats_ref):
    u = u_ref[0] + u_ref[1]
    den = den_ref[0] + den_ref[1]
    den = jnp.where(den == 0.0, 1.0, den)
    h_new = u / den[:, None]
    mu = jnp.mean(h_new, axis=0)
    var = jnp.mean((h_new - mu) ** 2, axis=0)
    inv = gh_ref[...] / jnp.sqrt(var + 1e-5)
    hout_ref[...] = jnp.maximum((h_new - mu) * inv + bh_ref[...], 0.0)
    cs = jnp.sum(csum_ref[...], axis=0)
    css = jnp.sum(cssq_ref[...], axis=0)
    mu_e = cs / E
    var_e = css / E - mu_e * mu_e
    scale = ge_ref[...] / jnp.sqrt(var_e + 1e-5)
    shift = be_ref[...] - mu_e * scale
    estats_ref[0, :] = scale
    estats_ref[1, :] = shift


def _finalize_e_body(ep_ref, st_ref, eout_ref):
    eout_ref[...] = jnp.maximum(ep_ref[...] * st_ref[0, :] + st_ref[1, :], 0.0)


def kernel(h, e, edge_index, W_h, W_e, W_p, b_p, W_a,
           gamma_h, beta_h, gamma_e, beta_e):
    f32 = jnp.float32
    sds = jax.ShapeDtypeStruct

    z_h, P_s, P_d, a_s2, a_d2, W_ep, w_ea = pl.pallas_call(
        _tables_body,
        out_shape=[sds((N, D), f32), sds((N, D), f32), sds((N, D), f32),
                   sds((N, 1), f32), sds((N, 1), f32),
                   sds((D, D), f32), sds((D, 1), f32)],
    )(h, W_h, W_e, W_p, W_a)

    BE = 4000
    P_e, a_e2 = pl.pallas_call(
        _edge_dense_body,
        grid=(E // BE,),
        in_specs=[pl.BlockSpec((BE, D), lambda i: (i, 0)),
                  pl.BlockSpec((D, D), lambda i: (0, 0)),
                  pl.BlockSpec((D, 1), lambda i: (0, 0))],
        out_specs=[pl.BlockSpec((BE, D), lambda i: (i, 0)),
                   pl.BlockSpec((BE, 1), lambda i: (i, 0))],
        out_shape=[sds((E, D), f32), sds((E, 1), f32)],
    )(e, W_ep, w_ea)

    src = edge_index[0]
    dst = edge_index[1]
    a_e = a_e2.reshape(E)
    a_s = a_s2.reshape(N)
    a_d = a_d2.reshape(N)

    mesh = plsc.VectorSubcoreMesh(core_axis_name="c", subcore_axis_name="s")
    sc_fn = pl.kernel(
        _sc_body,
        out_type=[sds((E, D), f32), sds((NC * N,), f32), sds((NC, N, D), f32),
                  sds((NW, D), f32), sds((NW, D), f32)],
        mesh=mesh,
        scratch_types=(
            [pltpu.VMEM((K,), jnp.int32),
             pltpu.VMEM((K,), jnp.int32),
             pltpu.VMEM((K,), f32),
             pltpu.VMEM((K,), f32),
             pltpu.VMEM((K,), f32),
             pltpu.VMEM((K,), f32),
             pltpu.VMEM((K, D), f32),
             pltpu.VMEM((K, D), f32),
             pltpu.VMEM((K, D), f32),
             pltpu.VMEM((K, D), f32),
             ] * 2
            + [
                pltpu.VMEM((D,), f32),
                pltpu.VMEM((D,), f32),
                pltpu.VMEM((DEN_PER_FLUSH,), f32),
                pltpu.VMEM((ZROWS, D), f32),
                pltpu.VMEM_SHARED((N, D), f32),
                pltpu.VMEM_SHARED((N,), f32),
            ]
            + [pltpu.SemaphoreType.DMA] * 8
        ),
    )
    e_proj, den2, u2, csum, cssq = sc_fn(
        src, dst, a_e, P_e, z_h, P_s, P_d, a_s, a_d)
    den2 = den2.reshape(NC, N)

    h_out, estats = pl.pallas_call(
        _finalize_h_body,
        out_shape=[sds((N, D), f32), sds((2, D), f32)],
    )(u2, den2, csum, cssq, gamma_h, beta_h, gamma_e, beta_e)

    e_out = pl.pallas_call(
        _finalize_e_body,
        grid=(E // BE,),
        in_specs=[pl.BlockSpec((BE, D), lambda i: (i, 0)),
                  pl.BlockSpec((2, D), lambda i: (0, 0))],
        out_specs=pl.BlockSpec((BE, D), lambda i: (i, 0)),
        out_shape=sds((E, D), f32),
    )(e_proj, estats)

    return h_out, e_out
